# baseline (speedup 1.0000x reference)
import functools

import jax
import jax.numpy as jnp
from jax import lax
from jax.experimental import pallas as pl
from jax.experimental.pallas import tpu as pltpu
from jax.experimental.pallas import tpu_sc as plsc

MAX_LEN = 3000
EMBED_DIM = 32
BATCH = 16384
SEQ = 200

NW = 32
PER_W = BATCH // NW
NB = 4
CHUNK = NB * SEQ
PIECE = 100
K = CHUNK // PIECE
NCHUNK = PER_W // NB
NBUF = 2
NITER = NCHUNK // NBUF

assert PER_W % NB == 0 and NCHUNK % NBUF == 0 and SEQ % PIECE == 0


def _emb_body(idx_hbm, tab_hbm, out_hbm,
              idx_v, rows_v,
              sem_i0, sem_i1, sem_g0, sem_g1, sem_o0, sem_o1):
    sem_i = [sem_i0, sem_i1]
    sem_g = [sem_g0, sem_g1]
    sem_o = [sem_o0, sem_o1]

    wid = lax.axis_index("s") * 2 + lax.axis_index("c")
    w_row0 = wid * (PER_W * SEQ // PIECE)
    w_b0 = wid * PER_W

    def stage_idx(chunk, b):
        src = idx_hbm.at[pl.ds(w_row0 + chunk * K, K)]
        pltpu.async_copy(src, idx_v.at[b], sem_i[b])

    for b in range(NBUF):
        stage_idx(b, b)

    def loop_body(i, carry):
        for b in range(NBUF):
            chunk = i * NBUF + b
            pltpu.make_async_copy(idx_hbm.at[pl.ds(0, K)], idx_v.at[b],
                                  sem_i[b]).wait()

            @pl.when(i >= 1)
            def _wait_prev_out():
                pltpu.make_async_copy(rows_v.at[b],
                                      out_hbm.at[pl.ds(0, NB)],
                                      sem_o[b]).wait()

            for j in range(K):
                pltpu.async_copy(
                    tab_hbm.at[idx_v.at[b, j]],
                    rows_v.at[b, j // 2, pl.ds((j % 2) * PIECE, PIECE), :],
                    sem_g[b])
            for j in range(K):
                pltpu.make_async_copy(
                    tab_hbm.at[idx_v.at[b, j]],
                    rows_v.at[b, j // 2, pl.ds((j % 2) * PIECE, PIECE), :],
                    sem_g[b]).wait()

            @pl.when(i < NITER - 1)
            def _stage_next():
                stage_idx(chunk + NBUF, b)

            pltpu.async_copy(rows_v.at[b],
                             out_hbm.at[pl.ds(w_b0 + chunk * NB, NB)],
                             sem_o[b])
        return carry

    lax.fori_loop(0, NITER, loop_body, 0)

    for b in range(NBUF):
        pltpu.make_async_copy(rows_v.at[b], out_hbm.at[pl.ds(0, NB)],
                              sem_o[b]).wait()


@jax.jit
def _emb(idx100, table):
    mesh = plsc.VectorSubcoreMesh(core_axis_name="c", subcore_axis_name="s")
    f = functools.partial(
        pl.kernel,
        mesh=mesh,
        out_type=jax.ShapeDtypeStruct((BATCH, SEQ, EMBED_DIM), jnp.float32),
        scratch_types=[
            pltpu.VMEM((NBUF, K, PIECE), jnp.int32),
            pltpu.VMEM((NBUF, NB, SEQ, EMBED_DIM), jnp.float32),
            pltpu.SemaphoreType.DMA,
            pltpu.SemaphoreType.DMA,
            pltpu.SemaphoreType.DMA,
            pltpu.SemaphoreType.DMA,
            pltpu.SemaphoreType.DMA,
            pltpu.SemaphoreType.DMA,
        ],
        compiler_params=pltpu.CompilerParams(use_tc_tiling_on_sc=False),
    )(_emb_body)
    return f(idx100, table)


def kernel(residue_idx, embed_weight):
    idx = residue_idx.astype(jnp.int32).reshape(BATCH * SEQ // PIECE, PIECE)
    return _emb(idx, embed_weight)


# speedup vs baseline: 2.3580x; 2.3580x over previous
import functools

import jax
import jax.numpy as jnp
from jax import lax
from jax.experimental import pallas as pl
from jax.experimental.pallas import tpu as pltpu
from jax.experimental.pallas import tpu_sc as plsc

MAX_LEN = 3000
EMBED_DIM = 32
BATCH = 16384
SEQ = 200

NW = 32
PER_W = BATCH // NW
BT = PER_W // 128
DT = EMBED_DIM // 8
NBUF = 2
L = 16
ROW = DT * (BATCH // 128) * 1024
DT_STRIDE = (BATCH // 128) * 1024
WBUF = BT * 1024


def _emb_body(idx_hbm, tab_hbm, out_hbm, tab_v, idx_v, buf_v,
              sem_i0, sem_i1, sem_o0, sem_o1, sem_t):
    sem_i = [sem_i0, sem_i1]
    sem_o = [sem_o0, sem_o1]

    wid = lax.axis_index("s") * 2 + lax.axis_index("c")
    b0 = wid * PER_W

    pltpu.async_copy(tab_hbm, tab_v, sem_t).wait()

    def stage_idx(s, slot):
        pltpu.async_copy(idx_hbm.at[pl.ds(s, 1), pl.ds(b0, PER_W)],
                         idx_v.at[slot], sem_i[slot])

    for slot in range(NBUF):
        stage_idx(slot, slot)

    def out_dst(s, dt):
        return out_hbm.at[pl.ds(s, 1),
                          pl.ds(dt * DT_STRIDE + wid * WBUF, WBUF)]

    def loop_body(s, carry):
        for slot in range(NBUF):
            pltpu.make_async_copy(idx_hbm.at[pl.ds(0, 1), pl.ds(0, PER_W)],
                                  idx_v.at[slot], sem_i[slot]).wait()

            @pl.when(s >= NBUF)
            def _wait_prev():
                for dt in range(DT):
                    pltpu.make_async_copy(buf_v.at[slot, dt], out_dst(0, dt),
                                          sem_o[slot]).wait()

            def gather_group(i, c):
                ids = idx_v[slot, 0, pl.ds(i * L, L)]
                base = (i % (128 // L)) * L + (i // (128 // L)) * 1024
                for d in range(EMBED_DIM):
                    vals = plsc.load_gather(tab_v, [ids + d * MAX_LEN])
                    buf_v[slot, d // 8, 0, pl.ds(base + (d % 8) * 128, L)] = vals
                return c

            lax.fori_loop(0, PER_W // L, gather_group, 0)

            @pl.when(s + slot + NBUF < SEQ)
            def _stage_next():
                stage_idx(s + slot + NBUF, slot)

            for dt in range(DT):
                pltpu.async_copy(buf_v.at[slot, dt], out_dst(s + slot, dt),
                                 sem_o[slot])
        return carry

    lax.fori_loop(0, SEQ // NBUF, lambda i, c: loop_body(i * NBUF, c), 0)

    for slot in range(NBUF):
        for dt in range(DT):
            pltpu.make_async_copy(buf_v.at[slot, dt], out_dst(0, dt),
                                  sem_o[slot]).wait()


@jax.jit
def _emb(idx_t, tab_t):
    mesh = plsc.VectorSubcoreMesh(core_axis_name="c", subcore_axis_name="s")
    f = functools.partial(
        pl.kernel,
        mesh=mesh,
        out_type=jax.ShapeDtypeStruct((SEQ, ROW), jnp.float32),
        scratch_types=[
            pltpu.VMEM((EMBED_DIM * MAX_LEN,), jnp.float32),
            pltpu.VMEM((NBUF, 1, PER_W), jnp.int32),
            pltpu.VMEM((NBUF, DT, 1, WBUF), jnp.float32),
            pltpu.SemaphoreType.DMA,
            pltpu.SemaphoreType.DMA,
            pltpu.SemaphoreType.DMA,
            pltpu.SemaphoreType.DMA,
            pltpu.SemaphoreType.DMA,
        ],
        compiler_params=pltpu.CompilerParams(use_tc_tiling_on_sc=False,
                                             needs_layout_passes=False),
    )(_emb_body)
    return f(idx_t, tab_t)


def kernel(residue_idx, embed_weight):
    idx_t = residue_idx.astype(jnp.int32).T
    tab_t = embed_weight.T.reshape(-1)
    out2 = _emb(idx_t, tab_t)
    out5 = out2.reshape(SEQ, DT, BATCH // 128, 8, 128)
    return out5.transpose(2, 4, 0, 1, 3).reshape(BATCH, SEQ, EMBED_DIM)


# speedup vs baseline: 7.2047x; 3.0554x over previous
import functools

import jax
import jax.numpy as jnp
from jax import lax
from jax.experimental import pallas as pl
from jax.experimental.pallas import tpu as pltpu
from jax.experimental.pallas import tpu_sc as plsc

MAX_LEN = 3000
EMBED_DIM = 32
BATCH = 16384
SEQ = 200

NW = 32
PER_W = BATCH // NW
BT = PER_W // 128
DT = EMBED_DIM // 8
NBUF = 2
L = 16
ROW = DT * (BATCH // 128) * 1024
DT_STRIDE = (BATCH // 128) * 1024
WBUF = BT * 1024


def _emb_body(idx_hbm, tab_hbm, out_hbm, tab_v, idx_v, buf_v,
              sem_i0, sem_i1, sem_o0, sem_o1, sem_t):
    sem_i = [sem_i0, sem_i1]
    sem_o = [sem_o0, sem_o1]

    wid = lax.axis_index("s") * 2 + lax.axis_index("c")
    b0 = wid * PER_W

    pltpu.async_copy(tab_hbm, tab_v, sem_t).wait()

    def stage_idx(s, slot):
        pltpu.async_copy(idx_hbm.at[pl.ds(s, 1), pl.ds(b0, PER_W)],
                         idx_v.at[slot], sem_i[slot])

    for slot in range(NBUF):
        stage_idx(slot, slot)

    def out_dst(s, dt):
        return out_hbm.at[pl.ds(s, 1),
                          pl.ds(dt * DT_STRIDE + wid * WBUF, WBUF)]

    def loop_body(s, carry):
        for slot in range(NBUF):
            pltpu.make_async_copy(idx_hbm.at[pl.ds(0, 1), pl.ds(0, PER_W)],
                                  idx_v.at[slot], sem_i[slot]).wait()

            @pl.when(s >= NBUF)
            def _wait_prev():
                for dt in range(DT):
                    pltpu.make_async_copy(buf_v.at[slot, dt], out_dst(0, dt),
                                          sem_o[slot]).wait()

            def gather_group(i, c):
                ids = idx_v[slot, 0, pl.ds(i * L, L)]
                base = (i % (128 // L)) * L + (i // (128 // L)) * 1024
                for dt in range(DT):
                    ds8 = range(dt * 8, dt * 8 + 8)
                    vals = [plsc.load_gather(tab_v, [ids + d * MAX_LEN])
                            for d in ds8]
                    for k, d in enumerate(ds8):
                        buf_v[slot, dt, 0,
                              pl.ds(base + (d % 8) * 128, L)] = vals[k]
                return c

            lax.fori_loop(0, PER_W // L, gather_group, 0)

            @pl.when(s + slot + NBUF < SEQ)
            def _stage_next():
                stage_idx(s + slot + NBUF, slot)

            for dt in range(DT):
                pltpu.async_copy(buf_v.at[slot, dt], out_dst(s + slot, dt),
                                 sem_o[slot])
        return carry

    lax.fori_loop(0, SEQ // NBUF, lambda i, c: loop_body(i * NBUF, c), 0)

    for slot in range(NBUF):
        for dt in range(DT):
            pltpu.make_async_copy(buf_v.at[slot, dt], out_dst(0, dt),
                                  sem_o[slot]).wait()


@jax.jit
def _emb(idx_t, tab_t):
    mesh = plsc.VectorSubcoreMesh(core_axis_name="c", subcore_axis_name="s")
    f = functools.partial(
        pl.kernel,
        mesh=mesh,
        out_type=jax.ShapeDtypeStruct((SEQ, ROW), jnp.float32),
        scratch_types=[
            pltpu.VMEM((EMBED_DIM * MAX_LEN,), jnp.float32),
            pltpu.VMEM((NBUF, 1, PER_W), jnp.int32),
            pltpu.VMEM((NBUF, DT, 1, WBUF), jnp.float32),
            pltpu.SemaphoreType.DMA,
            pltpu.SemaphoreType.DMA,
            pltpu.SemaphoreType.DMA,
            pltpu.SemaphoreType.DMA,
            pltpu.SemaphoreType.DMA,
        ],
        compiler_params=pltpu.CompilerParams(use_tc_tiling_on_sc=False,
                                             needs_layout_passes=False),
    )(_emb_body)
    return f(idx_t, tab_t)


def kernel(residue_idx, embed_weight):
    idx_t = residue_idx.astype(jnp.int32).T
    tab_t = embed_weight.T.reshape(-1)
    out2 = _emb(idx_t, tab_t)
    out5 = out2.reshape(SEQ, DT, BATCH // 128, 8, 128)
    return out5.transpose(2, 4, 0, 1, 3).reshape(BATCH, SEQ, EMBED_DIM)


# speedup vs baseline: 7.2601x; 1.0077x over previous
import functools

import jax
import jax.numpy as jnp
from jax import lax
from jax.experimental import pallas as pl
from jax.experimental.pallas import tpu as pltpu
from jax.experimental.pallas import tpu_sc as plsc

MAX_LEN = 3000
EMBED_DIM = 32
BATCH = 16384
SEQ = 200

NW = 32
PER_W = BATCH // NW
BT = PER_W // 128
DT = EMBED_DIM // 8
NBUF = 2
L = 16
DT_STRIDE = (BATCH // 128) * 1024
WBUF = BT * 1024
UNROLL = 2


def _emb_body(idx_hbm, tab_hbm, out_hbm, tab_v, idx_v, buf_v,
              sem_i0, sem_i1, sem_o0, sem_o1, sem_t):
    sem_i = [sem_i0, sem_i1]
    sem_o = [sem_o0, sem_o1]

    wid = lax.axis_index("s") * 2 + lax.axis_index("c")
    b0 = wid * PER_W

    pltpu.async_copy(tab_hbm, tab_v, sem_t).wait()

    def stage_idx(s, slot):
        pltpu.async_copy(idx_hbm.at[pl.ds(s, 1), pl.ds(b0, PER_W)],
                         idx_v.at[slot], sem_i[slot])

    for slot in range(NBUF):
        stage_idx(slot, slot)

    def out_dst(s):
        return out_hbm.at[pl.ds(s, 1), :, pl.ds(wid * WBUF, WBUF)]

    def loop_body(s, carry):
        for slot in range(NBUF):
            pltpu.make_async_copy(idx_hbm.at[pl.ds(0, 1), pl.ds(0, PER_W)],
                                  idx_v.at[slot], sem_i[slot]).wait()

            @pl.when(s >= NBUF)
            def _wait_prev():
                pltpu.make_async_copy(buf_v.at[slot], out_dst(0),
                                      sem_o[slot]).wait()

            def gather_group(i2, c):
                for u in range(UNROLL):
                    i = i2 * UNROLL + u
                    ids = idx_v[slot, 0, pl.ds(i * L, L)]
                    base = (i % (128 // L)) * L + (i // (128 // L)) * 1024
                    for dt in range(DT):
                        ds8 = range(dt * 8, dt * 8 + 8)
                        vals = [plsc.load_gather(tab_v, [ids + d * MAX_LEN])
                                for d in ds8]
                        for k, d in enumerate(ds8):
                            buf_v[slot, 0, dt,
                                  pl.ds(base + (d % 8) * 128, L)] = vals[k]
                return c

            lax.fori_loop(0, PER_W // L // UNROLL, gather_group, 0)

            @pl.when(s + slot + NBUF < SEQ)
            def _stage_next():
                stage_idx(s + slot + NBUF, slot)

            pltpu.async_copy(buf_v.at[slot], out_dst(s + slot), sem_o[slot])
        return carry

    lax.fori_loop(0, SEQ // NBUF, lambda i, c: loop_body(i * NBUF, c), 0)

    for slot in range(NBUF):
        pltpu.make_async_copy(buf_v.at[slot], out_dst(0), sem_o[slot]).wait()


@jax.jit
def _emb(idx_t, tab_t):
    mesh = plsc.VectorSubcoreMesh(core_axis_name="c", subcore_axis_name="s")
    f = functools.partial(
        pl.kernel,
        mesh=mesh,
        out_type=jax.ShapeDtypeStruct((SEQ, DT, DT_STRIDE), jnp.float32),
        scratch_types=[
            pltpu.VMEM((EMBED_DIM * MAX_LEN,), jnp.float32),
            pltpu.VMEM((NBUF, 1, PER_W), jnp.int32),
            pltpu.VMEM((NBUF, 1, DT, WBUF), jnp.float32),
            pltpu.SemaphoreType.DMA,
            pltpu.SemaphoreType.DMA,
            pltpu.SemaphoreType.DMA,
            pltpu.SemaphoreType.DMA,
            pltpu.SemaphoreType.DMA,
        ],
        compiler_params=pltpu.CompilerParams(use_tc_tiling_on_sc=False,
                                             needs_layout_passes=False),
    )(_emb_body)
    return f(idx_t, tab_t)


def kernel(residue_idx, embed_weight):
    idx_t = residue_idx.astype(jnp.int32).T
    tab_t = embed_weight.T.reshape(-1)
    out3 = _emb(idx_t, tab_t)
    out5 = out3.reshape(SEQ, DT, BATCH // 128, 8, 128)
    return out5.transpose(2, 4, 0, 1, 3).reshape(BATCH, SEQ, EMBED_DIM)


# speedup vs baseline: 7.4392x; 1.0247x over previous
import functools

import jax
import jax.numpy as jnp
from jax import lax
from jax.experimental import pallas as pl
from jax.experimental.pallas import tpu as pltpu
from jax.experimental.pallas import tpu_sc as plsc

MAX_LEN = 3000
EMBED_DIM = 32
BATCH = 16384
SEQ = 200

NW = 32
PER_W = BATCH // NW
BT = PER_W // 128
DT = EMBED_DIM // 8
NBUF = 2
L = 16
DT_STRIDE = (BATCH // 128) * 1024
WBUF = BT * 1024
UNROLL = 2


def _emb_body(idx_hbm, tab_hbm, out_hbm, tab_v, idx_v, buf_v,
              sem_i0, sem_i1, sem_o0, sem_o1, sem_t):
    sem_i = [sem_i0, sem_i1]
    sem_o = [sem_o0, sem_o1]

    wid = lax.axis_index("s") * 2 + lax.axis_index("c")
    b0 = wid * PER_W

    pltpu.async_copy(tab_hbm, tab_v, sem_t).wait()

    def stage_idx(s, slot):
        pltpu.async_copy(
            idx_hbm.at[pl.ds(s // 8, 1), pl.ds(wid * BT, BT),
                       pl.ds(s % 8, 1), :],
            idx_v.at[slot], sem_i[slot])

    for slot in range(NBUF):
        stage_idx(slot, slot)

    def out_dst(s):
        return out_hbm.at[pl.ds(s, 1), :, pl.ds(wid * WBUF, WBUF)]

    def loop_body(s, carry):
        for slot in range(NBUF):
            pltpu.make_async_copy(
                idx_hbm.at[pl.ds(0, 1), pl.ds(0, BT), pl.ds(0, 1), :],
                idx_v.at[slot], sem_i[slot]).wait()

            @pl.when(s >= NBUF)
            def _wait_prev():
                pltpu.make_async_copy(buf_v.at[slot], out_dst(0),
                                      sem_o[slot]).wait()

            def gather_group(i2, c):
                for u in range(UNROLL):
                    i = i2 * UNROLL + u
                    ids = idx_v[slot, 0, i // 8, 0, pl.ds((i % 8) * L, L)]
                    base = (i % (128 // L)) * L + (i // (128 // L)) * 1024
                    for dt in range(DT):
                        ds8 = range(dt * 8, dt * 8 + 8)
                        vals = [plsc.load_gather(tab_v, [ids + d * MAX_LEN])
                                for d in ds8]
                        for k, d in enumerate(ds8):
                            buf_v[slot, 0, dt,
                                  pl.ds(base + (d % 8) * 128, L)] = vals[k]
                return c

            lax.fori_loop(0, PER_W // L // UNROLL, gather_group, 0)

            @pl.when(s + slot + NBUF < SEQ)
            def _stage_next():
                stage_idx(s + slot + NBUF, slot)

            pltpu.async_copy(buf_v.at[slot], out_dst(s + slot), sem_o[slot])
        return carry

    lax.fori_loop(0, SEQ // NBUF, lambda i, c: loop_body(i * NBUF, c), 0)

    for slot in range(NBUF):
        pltpu.make_async_copy(buf_v.at[slot], out_dst(0), sem_o[slot]).wait()


@jax.jit
def _emb(idx_t, tab_t):
    mesh = plsc.VectorSubcoreMesh(core_axis_name="c", subcore_axis_name="s")
    f = functools.partial(
        pl.kernel,
        mesh=mesh,
        out_type=jax.ShapeDtypeStruct((SEQ, DT, DT_STRIDE), jnp.float32),
        scratch_types=[
            pltpu.VMEM((EMBED_DIM * MAX_LEN,), jnp.float32),
            pltpu.VMEM((NBUF, 1, BT, 1, 128), jnp.int32),
            pltpu.VMEM((NBUF, 1, DT, WBUF), jnp.float32),
            pltpu.SemaphoreType.DMA,
            pltpu.SemaphoreType.DMA,
            pltpu.SemaphoreType.DMA,
            pltpu.SemaphoreType.DMA,
            pltpu.SemaphoreType.DMA,
        ],
        compiler_params=pltpu.CompilerParams(use_tc_tiling_on_sc=False,
                                             needs_layout_passes=False),
    )(_emb_body)
    return f(idx_t, tab_t)


def kernel(residue_idx, embed_weight):
    idx_p = (residue_idx.astype(jnp.int32)
             .reshape(128, 128, 25, 8).transpose(2, 0, 3, 1))
    tab_t = embed_weight.T.reshape(-1)
    out3 = _emb(idx_p, tab_t)
    out5 = out3.reshape(SEQ, DT, BATCH // 128, 8, 128)
    return out5.transpose(2, 4, 0, 1, 3).reshape(BATCH, SEQ, EMBED_DIM)


# speedup vs baseline: 9.6261x; 1.2940x over previous
import functools

import jax
import jax.numpy as jnp
from jax import lax
from jax.experimental import pallas as pl
from jax.experimental.pallas import tpu as pltpu
from jax.experimental.pallas import tpu_sc as plsc

MAX_LEN = 3000
EMBED_DIM = 32
BATCH = 16384
SEQ = 200

NW = 32
PER_W = BATCH // NW
BT = PER_W // 128
DT = EMBED_DIM // 8
NBUF = 2
L = 16
DT_STRIDE = (BATCH // 128) * 1024
WBUF = BT * 1024
UNROLL = 4


def _emb_body(idx_hbm, tab_hbm, out_hbm, tab_v, idx_v, buf_v,
              sem_i0, sem_i1, sem_o0, sem_o1, sem_t):
    sem_i = [sem_i0, sem_i1]
    sem_o = [sem_o0, sem_o1]

    wid = lax.axis_index("s") * 2 + lax.axis_index("c")
    b0 = wid * PER_W

    pltpu.async_copy(tab_hbm, tab_v, sem_t).wait()

    def stage_idx(s, slot):
        pltpu.async_copy(
            idx_hbm.at[pl.ds(s // 8, 1), pl.ds(wid * BT, BT),
                       pl.ds(s % 8, 1), :],
            idx_v.at[slot], sem_i[slot])

    for slot in range(NBUF):
        stage_idx(slot, slot)

    def out_dst(s):
        return out_hbm.at[pl.ds(s, 1), :, pl.ds(wid * WBUF, WBUF)]

    def loop_body(s, carry):
        for slot in range(NBUF):
            pltpu.make_async_copy(
                idx_hbm.at[pl.ds(0, 1), pl.ds(0, BT), pl.ds(0, 1), :],
                idx_v.at[slot], sem_i[slot]).wait()

            @pl.when(s >= NBUF)
            def _wait_prev():
                pltpu.make_async_copy(buf_v.at[slot], out_dst(0),
                                      sem_o[slot]).wait()

            def gather_group(i2, c):
                pend = []
                for u in range(UNROLL):
                    i = i2 * UNROLL + u
                    ids = idx_v[slot, 0, i // 8, 0, pl.ds((i % 8) * L, L)]
                    base = (i % (128 // L)) * L + (i // (128 // L)) * 1024
                    for d in range(EMBED_DIM):
                        v = plsc.load_gather(tab_v, [ids + d * MAX_LEN])
                        if len(pend) >= 8:
                            pv, pdt, poff = pend.pop(0)
                            buf_v[slot, 0, pdt, pl.ds(poff, L)] = pv
                        pend.append((v, d // 8, base + (d % 8) * 128))
                for pv, pdt, poff in pend:
                    buf_v[slot, 0, pdt, pl.ds(poff, L)] = pv
                return c

            lax.fori_loop(0, PER_W // L // UNROLL, gather_group, 0)

            @pl.when(s + slot + NBUF < SEQ)
            def _stage_next():
                stage_idx(s + slot + NBUF, slot)

            pltpu.async_copy(buf_v.at[slot], out_dst(s + slot), sem_o[slot])
        return carry

    lax.fori_loop(0, SEQ // NBUF, lambda i, c: loop_body(i * NBUF, c), 0)

    for slot in range(NBUF):
        pltpu.make_async_copy(buf_v.at[slot], out_dst(0), sem_o[slot]).wait()


@jax.jit
def _emb(idx_t, tab_t):
    mesh = plsc.VectorSubcoreMesh(core_axis_name="c", subcore_axis_name="s")
    f = functools.partial(
        pl.kernel,
        mesh=mesh,
        out_type=jax.ShapeDtypeStruct((SEQ, DT, DT_STRIDE), jnp.float32),
        scratch_types=[
            pltpu.VMEM((EMBED_DIM * MAX_LEN,), jnp.float32),
            pltpu.VMEM((NBUF, 1, BT, 1, 128), jnp.int32),
            pltpu.VMEM((NBUF, 1, DT, WBUF), jnp.float32),
            pltpu.SemaphoreType.DMA,
            pltpu.SemaphoreType.DMA,
            pltpu.SemaphoreType.DMA,
            pltpu.SemaphoreType.DMA,
            pltpu.SemaphoreType.DMA,
        ],
        compiler_params=pltpu.CompilerParams(use_tc_tiling_on_sc=False,
                                             needs_layout_passes=False),
    )(_emb_body)
    return f(idx_t, tab_t)


def kernel(residue_idx, embed_weight):
    idx_p = (residue_idx.astype(jnp.int32)
             .reshape(128, 128, 25, 8).transpose(2, 0, 3, 1))
    tab_t = embed_weight.T.reshape(-1)
    out3 = _emb(idx_p, tab_t)
    out5 = out3.reshape(SEQ, DT, BATCH // 128, 8, 128)
    return out5.transpose(2, 4, 0, 1, 3).reshape(BATCH, SEQ, EMBED_DIM)


# speedup vs baseline: 10.0698x; 1.0461x over previous
import functools

import jax
import jax.numpy as jnp
from jax import lax
from jax.experimental import pallas as pl
from jax.experimental.pallas import tpu as pltpu
from jax.experimental.pallas import tpu_sc as plsc

MAX_LEN = 3000
EMBED_DIM = 32
BATCH = 16384
SEQ = 200

NW = 32
PER_W = BATCH // NW
BT = PER_W // 128
DT = EMBED_DIM // 8
NBUF = 2
L = 16
DT_STRIDE = (BATCH // 128) * 1024
WBUF = BT * 1024
UNROLL = 4


def _emb_body(idx_hbm, tab_hbm, out_hbm, tab_v, idx_v, buf_v,
              sem_i0, sem_i1, sem_o0, sem_o1, sem_t):
    sem_i = [sem_i0, sem_i1]
    sem_o = [sem_o0, sem_o1]

    wid = lax.axis_index("s") * 2 + lax.axis_index("c")
    b0 = wid * PER_W

    pltpu.async_copy(tab_hbm, tab_v, sem_t).wait()

    def stage_idx(s, slot):
        pltpu.async_copy(
            idx_hbm.at[pl.ds(s // 8, 1), pl.ds(wid * BT, BT),
                       pl.ds(s % 8, 1), :],
            idx_v.at[slot], sem_i[slot])

    for slot in range(NBUF):
        stage_idx(slot, slot)

    def out_dst(s):
        return out_hbm.at[pl.ds(s, 1), :, pl.ds(wid * WBUF, WBUF)]

    def loop_body(s, carry):
        for slot in range(NBUF):
            pltpu.make_async_copy(
                idx_hbm.at[pl.ds(0, 1), pl.ds(0, BT), pl.ds(0, 1), :],
                idx_v.at[slot], sem_i[slot]).wait()

            @pl.when(s >= NBUF)
            def _wait_prev():
                pltpu.make_async_copy(buf_v.at[slot], out_dst(0),
                                      sem_o[slot]).wait()

            def gather_group(i2, c):
                pend = []
                ids_list = [
                    idx_v[slot, 0, (i2 * UNROLL + u) // 8, 0,
                          pl.ds(((i2 * UNROLL + u) % 8) * L, L)]
                    for u in range(UNROLL)]
                for u in range(UNROLL):
                    i = i2 * UNROLL + u
                    ids = ids_list[u]
                    base = (i % (128 // L)) * L + (i // (128 // L)) * 1024
                    for d in range(EMBED_DIM):
                        v = plsc.load_gather(tab_v, [ids + d * MAX_LEN])
                        if len(pend) >= 8:
                            pv, pdt, poff = pend.pop(0)
                            buf_v[slot, 0, pdt, pl.ds(poff, L)] = pv
                        pend.append((v, d // 8, base + (d % 8) * 128))
                for pv, pdt, poff in pend:
                    buf_v[slot, 0, pdt, pl.ds(poff, L)] = pv
                return c

            lax.fori_loop(0, PER_W // L // UNROLL, gather_group, 0)

            @pl.when(s + slot + NBUF < SEQ)
            def _stage_next():
                stage_idx(s + slot + NBUF, slot)

            pltpu.async_copy(buf_v.at[slot], out_dst(s + slot), sem_o[slot])
        return carry

    lax.fori_loop(0, SEQ // NBUF, lambda i, c: loop_body(i * NBUF, c), 0)

    for slot in range(NBUF):
        pltpu.make_async_copy(buf_v.at[slot], out_dst(0), sem_o[slot]).wait()


@jax.jit
def _emb(idx_t, tab_t):
    mesh = plsc.VectorSubcoreMesh(core_axis_name="c", subcore_axis_name="s")
    f = functools.partial(
        pl.kernel,
        mesh=mesh,
        out_type=jax.ShapeDtypeStruct((SEQ, DT, DT_STRIDE), jnp.float32),
        scratch_types=[
            pltpu.VMEM((EMBED_DIM * MAX_LEN,), jnp.float32),
            pltpu.VMEM((NBUF, 1, BT, 1, 128), jnp.int32),
            pltpu.VMEM((NBUF, 1, DT, WBUF), jnp.float32),
            pltpu.SemaphoreType.DMA,
            pltpu.SemaphoreType.DMA,
            pltpu.SemaphoreType.DMA,
            pltpu.SemaphoreType.DMA,
            pltpu.SemaphoreType.DMA,
        ],
        compiler_params=pltpu.CompilerParams(use_tc_tiling_on_sc=False,
                                             needs_layout_passes=False),
    )(_emb_body)
    return f(idx_t, tab_t)


def kernel(residue_idx, embed_weight):
    idx_p = (residue_idx.astype(jnp.int32)
             .reshape(128, 128, 25, 8).transpose(2, 0, 3, 1))
    tab_t = embed_weight.T.reshape(-1)
    out3 = _emb(idx_p, tab_t)
    out5 = out3.reshape(SEQ, DT, BATCH // 128, 8, 128)
    return out5.transpose(2, 4, 0, 1, 3).reshape(BATCH, SEQ, EMBED_DIM)


# speedup vs baseline: 10.1421x; 1.0072x over previous
import functools

import jax
import jax.numpy as jnp
from jax import lax
from jax.experimental import pallas as pl
from jax.experimental.pallas import tpu as pltpu
from jax.experimental.pallas import tpu_sc as plsc

MAX_LEN = 3000
EMBED_DIM = 32
BATCH = 16384
SEQ = 200

NW = 32
PER_W = BATCH // NW
BT = PER_W // 128
DT = EMBED_DIM // 8
NBUF = 2
L = 16
DT_STRIDE = (BATCH // 128) * 1024
WBUF = BT * 1024
UNROLL = 8


def _emb_body(idx_hbm, tab_hbm, out_hbm, tab_v, idx_v, buf_v,
              sem_i0, sem_i1, sem_o0, sem_o1, sem_t):
    sem_i = [sem_i0, sem_i1]
    sem_o = [sem_o0, sem_o1]

    wid = lax.axis_index("s") * 2 + lax.axis_index("c")
    b0 = wid * PER_W

    pltpu.async_copy(tab_hbm, tab_v, sem_t).wait()

    def stage_idx(s, slot):
        pltpu.async_copy(
            idx_hbm.at[pl.ds(s // 8, 1), pl.ds(wid * BT, BT),
                       pl.ds(s % 8, 1), :],
            idx_v.at[slot], sem_i[slot])

    for slot in range(NBUF):
        stage_idx(slot, slot)

    def out_dst(s):
        return out_hbm.at[pl.ds(s, 1), :, pl.ds(wid * WBUF, WBUF)]

    def loop_body(s, carry):
        for slot in range(NBUF):
            pltpu.make_async_copy(
                idx_hbm.at[pl.ds(0, 1), pl.ds(0, BT), pl.ds(0, 1), :],
                idx_v.at[slot], sem_i[slot]).wait()

            @pl.when(s >= NBUF)
            def _wait_prev():
                pltpu.make_async_copy(buf_v.at[slot], out_dst(0),
                                      sem_o[slot]).wait()

            def gather_group(i2, c):
                pend = []
                ids_list = [
                    idx_v[slot, 0, (i2 * UNROLL + u) // 8, 0,
                          pl.ds(((i2 * UNROLL + u) % 8) * L, L)]
                    for u in range(UNROLL)]
                for u in range(UNROLL):
                    i = i2 * UNROLL + u
                    ids = ids_list[u]
                    base = (i % (128 // L)) * L + (i // (128 // L)) * 1024
                    for d in range(EMBED_DIM):
                        v = plsc.load_gather(tab_v, [ids + d * MAX_LEN])
                        if len(pend) >= 8:
                            pv, pdt, poff = pend.pop(0)
                            buf_v[slot, 0, pdt, pl.ds(poff, L)] = pv
                        pend.append((v, d // 8, base + (d % 8) * 128))
                for pv, pdt, poff in pend:
                    buf_v[slot, 0, pdt, pl.ds(poff, L)] = pv
                return c

            lax.fori_loop(0, PER_W // L // UNROLL, gather_group, 0)

            @pl.when(s + slot + NBUF < SEQ)
            def _stage_next():
                stage_idx(s + slot + NBUF, slot)

            pltpu.async_copy(buf_v.at[slot], out_dst(s + slot), sem_o[slot])
        return carry

    lax.fori_loop(0, SEQ // NBUF, lambda i, c: loop_body(i * NBUF, c), 0)

    for slot in range(NBUF):
        pltpu.make_async_copy(buf_v.at[slot], out_dst(0), sem_o[slot]).wait()


@jax.jit
def _emb(idx_t, tab_t):
    mesh = plsc.VectorSubcoreMesh(core_axis_name="c", subcore_axis_name="s")
    f = functools.partial(
        pl.kernel,
        mesh=mesh,
        out_type=jax.ShapeDtypeStruct((SEQ, DT, DT_STRIDE), jnp.float32),
        scratch_types=[
            pltpu.VMEM((EMBED_DIM * MAX_LEN,), jnp.float32),
            pltpu.VMEM((NBUF, 1, BT, 1, 128), jnp.int32),
            pltpu.VMEM((NBUF, 1, DT, WBUF), jnp.float32),
            pltpu.SemaphoreType.DMA,
            pltpu.SemaphoreType.DMA,
            pltpu.SemaphoreType.DMA,
            pltpu.SemaphoreType.DMA,
            pltpu.SemaphoreType.DMA,
        ],
        compiler_params=pltpu.CompilerParams(use_tc_tiling_on_sc=False,
                                             needs_layout_passes=False),
    )(_emb_body)
    return f(idx_t, tab_t)


def kernel(residue_idx, embed_weight):
    idx_p = (residue_idx.astype(jnp.int32)
             .reshape(128, 128, 25, 8).transpose(2, 0, 3, 1))
    tab_t = embed_weight.T.reshape(-1)
    out3 = _emb(idx_p, tab_t)
    out5 = out3.reshape(SEQ, DT, BATCH // 128, 8, 128)
    return out5.transpose(2, 4, 0, 1, 3).reshape(BATCH, SEQ, EMBED_DIM)


# speedup vs baseline: 11.2992x; 1.1141x over previous
import functools

import jax
import jax.numpy as jnp
from jax import lax
from jax.experimental import pallas as pl
from jax.experimental.pallas import tpu as pltpu
from jax.experimental.pallas import tpu_sc as plsc

MAX_LEN = 3000
EMBED_DIM = 32
BATCH = 16384
SEQ = 200

DT = EMBED_DIM // 8
NG = 8
PER_W = BATCH // NG
BT = PER_W // 128
NBUF = 4
L = 16
DT_STRIDE = (BATCH // 128) * 1024
WBUF = BT * 1024
NGROUP = PER_W // L
UNROLL = 8


def _emb_body(idx_hbm, tab_hbm, out_hbm, tab_v, idx_v, buf_v,
              sem_i, sem_o, sem_t):
    wid = lax.axis_index("s") * 2 + lax.axis_index("c")
    dt_w = wid // NG
    bw = wid % NG

    pltpu.async_copy(tab_hbm.at[pl.ds(dt_w * 8 * MAX_LEN, 8 * MAX_LEN)],
                     tab_v, sem_t).wait()

    def stage_idx(s, slot):
        pltpu.async_copy(
            idx_hbm.at[pl.ds(s // 8, 1), pl.ds(bw * BT, BT),
                       pl.ds(s % 8, 1), :],
            idx_v.at[slot], sem_i[slot])

    for slot in range(NBUF):
        stage_idx(slot, slot)

    def out_dst(s):
        return out_hbm.at[pl.ds(s, 1),
                          pl.ds(dt_w * DT_STRIDE + bw * WBUF, WBUF)]

    def loop_body(s, carry):
        for slot in range(NBUF):
            pltpu.make_async_copy(
                idx_hbm.at[pl.ds(0, 1), pl.ds(0, BT), pl.ds(0, 1), :],
                idx_v.at[slot], sem_i[slot]).wait()

            @pl.when(s >= NBUF)
            def _wait_prev():
                pltpu.make_async_copy(buf_v.at[slot], out_dst(0),
                                      sem_o[slot]).wait()

            def gather_group(i2, c):
                pend = []
                ids_list = [
                    idx_v[slot, 0, (i2 * UNROLL + u) // 8, 0,
                          pl.ds(((i2 * UNROLL + u) % 8) * L, L)]
                    for u in range(UNROLL)]
                for u in range(UNROLL):
                    i = i2 * UNROLL + u
                    ids = ids_list[u]
                    base = (i % 8) * L + (i // 8) * 1024
                    for dl in range(8):
                        v = plsc.load_gather(tab_v, [ids + dl * MAX_LEN])
                        if len(pend) >= 8:
                            pv, poff = pend.pop(0)
                            buf_v[slot, 0, pl.ds(poff, L)] = pv
                        pend.append((v, base + dl * 128))
                for pv, poff in pend:
                    buf_v[slot, 0, pl.ds(poff, L)] = pv
                return c

            lax.fori_loop(0, NGROUP // UNROLL, gather_group, 0)

            @pl.when(s + slot + NBUF < SEQ)
            def _stage_next():
                stage_idx(s + slot + NBUF, slot)

            pltpu.async_copy(buf_v.at[slot], out_dst(s + slot), sem_o[slot])
        return carry

    lax.fori_loop(0, SEQ // NBUF, lambda i, c: loop_body(i * NBUF, c), 0)

    for slot in range(NBUF):
        pltpu.make_async_copy(buf_v.at[slot], out_dst(0), sem_o[slot]).wait()


@jax.jit
def _emb(idx_t, tab_t):
    mesh = plsc.VectorSubcoreMesh(core_axis_name="c", subcore_axis_name="s")
    f = functools.partial(
        pl.kernel,
        mesh=mesh,
        out_type=jax.ShapeDtypeStruct((SEQ, DT * DT_STRIDE), jnp.float32),
        scratch_types=[
            pltpu.VMEM((8 * MAX_LEN,), jnp.float32),
            pltpu.VMEM((NBUF, 1, BT, 1, 128), jnp.int32),
            pltpu.VMEM((NBUF, 1, WBUF), jnp.float32),
            [pltpu.SemaphoreType.DMA] * NBUF,
            [pltpu.SemaphoreType.DMA] * NBUF,
            pltpu.SemaphoreType.DMA,
        ],
        compiler_params=pltpu.CompilerParams(use_tc_tiling_on_sc=False,
                                             needs_layout_passes=False),
    )(_emb_body)
    return f(idx_t, tab_t)


def kernel(residue_idx, embed_weight):
    idx_p = (residue_idx.astype(jnp.int32)
             .reshape(128, 128, 25, 8).transpose(2, 0, 3, 1))
    tab_t = embed_weight.T.reshape(-1)
    out2 = _emb(idx_p, tab_t)
    out5 = out2.reshape(SEQ, DT, BATCH // 128, 8, 128)
    return out5.transpose(2, 4, 0, 1, 3).reshape(BATCH, SEQ, EMBED_DIM)
